# Initial kernel scaffold; baseline (speedup 1.0000x reference)
import jax
import jax.numpy as jnp
from jax.experimental import pallas as pl


def kernel(x, pe_table):
    raise NotImplementedError("write your pallas kernel here")


# speedup vs baseline: 3.4373x; 3.4373x over previous
import functools

import jax
import jax.numpy as jnp
from jax import lax
from jax.experimental import pallas as pl
from jax.experimental.pallas import tpu as pltpu
from jax.experimental.pallas import tpu_sc as plsc

_B, _S = 16384, 200
_D = 64
_TOTAL = _B * _S
_NC, _NS = 2, 16
_NW = _NC * _NS
_PER_W = _TOTAL // _NW
_CHUNK = 128
_NCHUNK = _PER_W // _CHUNK


def _emb_lookup(table, idx):
    mesh = plsc.VectorSubcoreMesh(core_axis_name="c", subcore_axis_name="s")

    @functools.partial(
        pl.kernel,
        mesh=mesh,
        out_type=jax.ShapeDtypeStruct((_TOTAL, _D), jnp.float32),
        compiler_params=pltpu.CompilerParams(use_tc_tiling_on_sc=False),
        scratch_types=[
            pltpu.VMEM((1, _CHUNK), jnp.int32),
            pltpu.VMEM((1, _CHUNK, _D), jnp.float32),
            pltpu.SemaphoreType.DMA,
        ],
    )
    def k(table_hbm, idx_hbm, out_hbm, idx_v, rows_v, gsem):
        wid = lax.axis_index("s") * _NC + lax.axis_index("c")
        base = wid * _PER_W

        def step(i, carry):
            off = base + i * _CHUNK
            pltpu.sync_copy(idx_hbm.at[pl.ds(off, _CHUNK)], idx_v.at[0])
            pltpu.async_copy(table_hbm.at[idx_v.at[0]], rows_v.at[0], gsem).wait()
            pltpu.sync_copy(rows_v.at[0], out_hbm.at[pl.ds(off, _CHUNK)])
            return carry

        lax.fori_loop(0, _NCHUNK, step, 0)

    return k(table, idx)


def kernel(x, pe_table):
    idx = x.reshape(_TOTAL)
    out = _emb_lookup(pe_table, idx)
    return out.reshape(_B, _S, _D)


# speedup vs baseline: 5.1006x; 1.4839x over previous
import functools

import jax
import jax.numpy as jnp
from jax import lax
from jax.experimental import pallas as pl
from jax.experimental.pallas import tpu as pltpu
from jax.experimental.pallas import tpu_sc as plsc

_B, _S = 16384, 200
_D = 64
_V = 500
_NC, _NS = 2, 16
_NW = _NC * _NS
_BS = _B // _NW
_L = 16
_NG = _BS // _L


def _lookup(table_t, x_t):
    mesh = plsc.VectorSubcoreMesh(core_axis_name="c", subcore_axis_name="s")

    @functools.partial(
        pl.kernel,
        mesh=mesh,
        out_type=jax.ShapeDtypeStruct((_S, _D, _B), jnp.float32),
        compiler_params=pltpu.CompilerParams(
            use_tc_tiling_on_sc=False, needs_layout_passes=False),
        scratch_types=[
            pltpu.VMEM((_D * _V,), jnp.float32),
            pltpu.VMEM((2, _BS), jnp.int32),
            pltpu.VMEM((2, _D, _BS), jnp.float32),
            pltpu.SemaphoreType.DMA,
            pltpu.SemaphoreType.DMA,
            pltpu.SemaphoreType.DMA,
            pltpu.SemaphoreType.DMA,
        ],
    )
    def k(tab_hbm, xt_hbm, out_hbm, tab_v, idx_v, blk_v, isem0, isem1,
          osem0, osem1):
        wid = lax.axis_index("s") * _NC + lax.axis_index("c")
        bs = wid * _BS

        pltpu.sync_copy(tab_hbm, tab_v)

        def idx_load(l, h):
            isem = isem0 if h == 0 else isem1
            return pltpu.async_copy(
                xt_hbm.at[l, pl.ds(bs, _BS)], idx_v.at[h], isem)

        def compute_block(h):
            def per_group(g, carry):
                idx_vec = idx_v[h, pl.ds(g * _L, _L)]
                for d in range(_D):
                    vals = plsc.load_gather(tab_v, [idx_vec + d * _V])
                    blk_v[h, d, pl.ds(g * _L, _L)] = vals
                return carry
            lax.fori_loop(0, _NG, per_group, 0)

        def write_block(l, h):
            osem = osem0 if h == 0 else osem1
            return pltpu.async_copy(
                blk_v.at[h], out_hbm.at[l, :, pl.ds(bs, _BS)], osem)

        def drain_write(h):
            osem = osem0 if h == 0 else osem1
            pltpu.make_async_copy(
                blk_v.at[h], out_hbm.at[0, :, pl.ds(bs, _BS)], osem).wait()

        def drain_idx(h):
            isem = isem0 if h == 0 else isem1
            pltpu.make_async_copy(
                xt_hbm.at[0, pl.ds(bs, _BS)], idx_v.at[h], isem).wait()

        idx_load(0, 0)
        idx_load(1, 1)

        def pair(p, carry):
            l0 = 2 * p
            for h in (0, 1):
                l = l0 + h
                drain_idx(h)

                @pl.when(l >= 2)
                def _():
                    drain_write(h)

                compute_block(h)
                write_block(l, h)

                @pl.when(l + 2 < _S)
                def _():
                    idx_load(l + 2, h)
            return carry

        lax.fori_loop(0, _S // 2, pair, 0)
        drain_write(0)
        drain_write(1)

    return k(table_t, x_t)


def kernel(x, pe_table):
    x_t = x.T
    table_t = pe_table.T.reshape(_D * _V)
    out_phys = _lookup(table_t, x_t)
    return jnp.transpose(out_phys, (2, 0, 1))


# speedup vs baseline: 17.0124x; 3.3354x over previous
import functools

import jax
import jax.numpy as jnp
from jax import lax
from jax.experimental import pallas as pl
from jax.experimental.pallas import tpu as pltpu
from jax.experimental.pallas import tpu_sc as plsc

_B, _S = 16384, 200
_D = 64
_V = 500
_VP = 512
_NC, _NS = 2, 16
_NW = _NC * _NS
_BS = _B // _NW
_L = 16
_NG = _BS // _L


def _lookup(table_t, x_t):
    mesh = plsc.VectorSubcoreMesh(core_axis_name="c", subcore_axis_name="s")

    @functools.partial(
        pl.kernel,
        mesh=mesh,
        out_type=jax.ShapeDtypeStruct((_S, _D, _B), jnp.float32),
        compiler_params=pltpu.CompilerParams(
            use_tc_tiling_on_sc=True, needs_layout_passes=False),
        scratch_types=[
            pltpu.VMEM((_D * _VP,), jnp.float32),
            pltpu.VMEM((2, _BS), jnp.int32),
            pltpu.VMEM((2, _D, _BS), jnp.float32),
            pltpu.SemaphoreType.DMA,
            pltpu.SemaphoreType.DMA,
            pltpu.SemaphoreType.DMA,
            pltpu.SemaphoreType.DMA,
        ],
    )
    def k(tab_hbm, xt_hbm, out_hbm, tab_v, idx_v, blk_v, isem0, isem1,
          osem0, osem1):
        wid = lax.axis_index("s") * _NC + lax.axis_index("c")
        bs = wid * _BS

        pltpu.sync_copy(tab_hbm, tab_v)

        def idx_load(l, h):
            isem = isem0 if h == 0 else isem1
            return pltpu.async_copy(
                xt_hbm.at[pl.ds(l * _B + bs, _BS)], idx_v.at[h], isem)

        def compute_block(h):
            def per_group(g, carry):
                idx_vec = idx_v[h, pl.ds(g * _L, _L)]
                for d0 in range(0, _D, 8):
                    vals = [
                        plsc.load_gather(tab_v, [idx_vec + (d0 + t) * _VP])
                        for t in range(8)
                    ]
                    for t in range(8):
                        blk_v[h, d0 + t, pl.ds(g * _L, _L)] = vals[t]
                return carry
            lax.fori_loop(0, _NG, per_group, 0)

        def write_block(l, h):
            osem = osem0 if h == 0 else osem1
            return pltpu.async_copy(
                blk_v.at[h], out_hbm.at[l, :, pl.ds(bs, _BS)], osem)

        def drain_write(h):
            osem = osem0 if h == 0 else osem1
            pltpu.make_async_copy(
                blk_v.at[h], out_hbm.at[0, :, pl.ds(bs, _BS)], osem).wait()

        def drain_idx(h):
            isem = isem0 if h == 0 else isem1
            pltpu.make_async_copy(
                xt_hbm.at[pl.ds(bs, _BS)], idx_v.at[h], isem).wait()

        idx_load(0, 0)
        idx_load(1, 1)

        def pair(p, carry):
            l0 = 2 * p
            for h in (0, 1):
                l = l0 + h
                drain_idx(h)

                @pl.when(l >= 2)
                def _():
                    drain_write(h)

                compute_block(h)
                write_block(l, h)

                @pl.when(l + 2 < _S)
                def _():
                    idx_load(l + 2, h)
            return carry

        lax.fori_loop(0, _S // 2, pair, 0)
        drain_write(0)
        drain_write(1)

    return k(table_t, x_t)


def kernel(x, pe_table):
    x_t = x.T.reshape(_S * _B)
    table_t = jnp.pad(pe_table.T, ((0, 0), (0, _VP - _V))).reshape(_D * _VP)
    out_phys = _lookup(table_t, x_t)
    return jnp.transpose(out_phys, (2, 0, 1))
